# baseline (speedup 1.0000x reference)
import functools

import jax
import jax.numpy as jnp
from jax import lax
from jax.experimental import pallas as pl
from jax.experimental.pallas import tpu as pltpu
from jax.experimental.pallas import tpu_sc as plsc

N_GRAPHS_ = 64
L_ = 16
BK_ = 128
CR_ = 20


@functools.partial(jax.jit, static_argnames=("n_edges", "n_nodes"))
def _sc_virial(d_rows, f_rows, edge_index, bpacked, *, n_edges, n_nodes):
    info = plsc.get_sparse_core_info()
    nc, ns = info.num_cores, info.num_subcores
    nw = nc * ns
    n_rows = n_edges // BK_
    assert n_rows * BK_ == n_edges and n_rows % CR_ == 0
    n_chunks = n_rows // CR_
    iters = (n_chunks + nw - 1) // nw
    cedges = CR_ * BK_
    nbins = N_GRAPHS_ * 9
    gstride = N_GRAPHS_ + 1
    n_words = (n_nodes + 1) // 2

    mesh = plsc.VectorSubcoreMesh(core_axis_name="c", subcore_axis_name="s")

    @functools.partial(
        pl.kernel,
        out_type=jax.ShapeDtypeStruct((nw * nbins,), jnp.float32),
        mesh=mesh,
        compiler_params=pltpu.CompilerParams(
            needs_layout_passes=False, use_tc_tiling_on_sc=False),
        scratch_types=[
            pltpu.VMEM((n_words,), jnp.int32),
            pltpu.VMEM((2, CR_, 4 * BK_), jnp.float32),
            pltpu.VMEM((2, CR_, 4 * BK_), jnp.float32),
            pltpu.VMEM((2, 2, cedges), jnp.int32),
            pltpu.VMEM((nbins,), jnp.float32),
            pltpu.SemaphoreType.DMA,
            pltpu.SemaphoreType.DMA,
        ] + [
            pltpu.VMEM((L_ * gstride,), jnp.float32)
            for _ in range(18)
        ],
    )
    def k(d_hbm, f_hbm, ei_hbm, b_hbm, out_hbm,
          btbl, dbuf, fbuf, ibuf, orow, sem0, sem1, *accs):
        wid = lax.axis_index("s") * nc + lax.axis_index("c")
        lanes = lax.iota(jnp.int32, L_)
        lane_base = lanes * gstride
        zf16 = jnp.zeros((L_,), jnp.float32)

        pltpu.sync_copy(b_hbm, btbl)

        def zero_body(i, _):
            for a in accs:
                a[pl.ds(i * L_, L_)] = zf16
            return 0

        lax.fori_loop(0, (L_ * gstride) // L_, zero_body, 0)

        def issue(i, slot, sem):
            cid = i * nw + wid

            @pl.when(cid < n_chunks)
            def _():
                pltpu.async_copy(d_hbm.at[pl.ds(cid * CR_, CR_)],
                                 dbuf.at[slot], sem)
                pltpu.async_copy(f_hbm.at[pl.ds(cid * CR_, CR_)],
                                 fbuf.at[slot], sem)
                pltpu.async_copy(ei_hbm.at[:, pl.ds(cid * cedges, cedges)],
                                 ibuf.at[slot], sem)

        def drain(i, slot, sem):
            cid = i * nw + wid

            @pl.when(cid < n_chunks)
            def _():
                pltpu.make_async_copy(d_hbm.at[pl.ds(cid * CR_, CR_)],
                                      dbuf.at[slot], sem).wait()
                pltpu.make_async_copy(f_hbm.at[pl.ds(cid * CR_, CR_)],
                                      fbuf.at[slot], sem).wait()
                pltpu.make_async_copy(ei_hbm.at[:, pl.ds(cid * cedges, cedges)],
                                      ibuf.at[slot], sem).wait()

        def process(i, slot):
            cid = i * nw + wid

            @pl.when(cid < n_chunks)
            def _():
                def row_body(r, _):
                    for j in range(BK_ // L_):
                        o = j * L_
                        dx = dbuf[slot, r, pl.ds(o, L_)]
                        dy = dbuf[slot, r, pl.ds(BK_ + o, L_)]
                        dz = dbuf[slot, r, pl.ds(2 * BK_ + o, L_)]
                        fx = fbuf[slot, r, pl.ds(o, L_)]
                        fy = fbuf[slot, r, pl.ds(BK_ + o, L_)]
                        fz = fbuf[slot, r, pl.ds(2 * BK_ + o, L_)]
                        i0 = ibuf[slot, 0, pl.ds(r * BK_ + o, L_)]
                        i1 = ibuf[slot, 1, pl.ds(r * BK_ + o, L_)]
                        w0 = plsc.load_gather(btbl, [lax.shift_right_logical(i0, 1)])
                        w1 = plsc.load_gather(btbl, [lax.shift_right_logical(i1, 1)])
                        g0 = lax.shift_right_logical(
                            w0, lax.shift_left(i0 & 1, 4)) & 0xFFFF
                        g1 = lax.shift_right_logical(
                            w1, lax.shift_left(i1 & 1, 4)) & 0xFFFF
                        b0 = lane_base + g0
                        b1 = lane_base + g1
                        c = 0
                        for dc in (dx, dy, dz):
                            for fc in (fx, fy, fz):
                                p = dc * fc
                                a0 = accs[2 * c]
                                a1 = accs[2 * c + 1]
                                plsc.store_scatter(
                                    a0, [b0], plsc.load_gather(a0, [b0]) + p)
                                plsc.store_scatter(
                                    a1, [b1], plsc.load_gather(a1, [b1]) + p)
                                c += 1
                    return 0

                lax.fori_loop(0, CR_, row_body, 0)

        issue(0, 0, sem0)

        def chunk_body(i, _):
            slot = lax.rem(i, 2)

            @pl.when(slot == 0)
            def _():
                issue(i + 1, 1, sem1)
                drain(i, 0, sem0)
                process(i, 0)

            @pl.when(slot == 1)
            def _():
                issue(i + 1, 0, sem0)
                drain(i, 1, sem1)
                process(i, 1)

            return 0

        lax.fori_loop(0, iters, chunk_body, 0)

        for c in range(9):
            def red_body(j, _, c=c):
                s = zf16
                for lane in range(L_):
                    off = lane * gstride + j * L_
                    s = s + accs[2 * c][pl.ds(off, L_)]                           + accs[2 * c + 1][pl.ds(off, L_)]
                orow[pl.ds(c * N_GRAPHS_ + j * L_, L_)] = s
                return 0

            lax.fori_loop(0, N_GRAPHS_ // L_, red_body, 0)
        pltpu.sync_copy(orow, out_hbm.at[pl.ds(wid * nbins, nbins)])

    return k(d_rows, f_rows, edge_index, bpacked)


def _pack_batch(batch):
    b = batch.astype(jnp.int32).reshape(-1, 2)
    return b[:, 0] | (b[:, 1] << 16)


def _to_block_rows(x):
    nb = x.shape[0] // BK_
    return jnp.pad(x.reshape(nb, BK_, 3), ((0, 0), (0, 0), (0, 1))) \
              .swapaxes(1, 2).reshape(nb, 4 * BK_)


def kernel(disp, pairwise_force, edge_index, batch):
    n_edges = disp.shape[0]
    n_nodes = batch.shape[0]
    partials = _sc_virial(
        _to_block_rows(disp.astype(jnp.float32)),
        _to_block_rows(pairwise_force.astype(jnp.float32)),
        edge_index.astype(jnp.int32),
        _pack_batch(batch),
        n_edges=n_edges, n_nodes=n_nodes)
    out9g = partials.reshape(-1, 9, N_GRAPHS_).sum(axis=0)
    return out9g.T.reshape(N_GRAPHS_, 3, 3)


# speedup vs baseline: 2.3376x; 2.3376x over previous
import functools

import jax
import jax.numpy as jnp
from jax import lax
from jax.experimental import pallas as pl
from jax.experimental.pallas import tpu as pltpu
from jax.experimental.pallas import tpu_sc as plsc

N_GRAPHS_ = 64
L_ = 16
BK_ = 128
CR_ = 20


@functools.partial(jax.jit, static_argnames=("n_edges", "n_nodes"))
def _sc_virial(d_rows, f_rows, edge_index, bpacked, *, n_edges, n_nodes):
    info = plsc.get_sparse_core_info()
    nc, ns = info.num_cores, info.num_subcores
    nw = nc * ns
    n_rows = n_edges // BK_
    assert n_rows * BK_ == n_edges and n_rows % CR_ == 0
    n_chunks = n_rows // CR_
    iters = (n_chunks + nw - 1) // nw
    cedges = CR_ * BK_
    nbins = N_GRAPHS_ * 9
    astride = nbins + 1
    n_words = (n_nodes + 1) // 2

    mesh = plsc.VectorSubcoreMesh(core_axis_name="c", subcore_axis_name="s")

    @functools.partial(
        pl.kernel,
        out_type=jax.ShapeDtypeStruct((nw * nbins,), jnp.float32),
        mesh=mesh,
        compiler_params=pltpu.CompilerParams(
            needs_layout_passes=False, use_tc_tiling_on_sc=False),
        scratch_types=[
            pltpu.VMEM((n_words,), jnp.int32),
            pltpu.VMEM((L_ * astride,), jnp.float32),
            pltpu.VMEM((L_ * astride,), jnp.float32),
            pltpu.VMEM((2, CR_, 4 * BK_), jnp.float32),
            pltpu.VMEM((2, CR_, 4 * BK_), jnp.float32),
            pltpu.VMEM((2, 2, cedges), jnp.int32),
            pltpu.VMEM((nbins,), jnp.float32),
            pltpu.SemaphoreType.DMA,
            pltpu.SemaphoreType.DMA,
        ],
    )
    def k(d_hbm, f_hbm, ei_hbm, b_hbm, out_hbm,
          btbl, acc, acc2, dbuf, fbuf, ibuf, orow, sem0, sem1):
        wid = lax.axis_index("s") * nc + lax.axis_index("c")
        lanes = lax.iota(jnp.int32, L_)
        lane_base = lanes * astride
        zf16 = jnp.zeros((L_,), jnp.float32)

        pltpu.sync_copy(b_hbm, btbl)

        def zero_body(i, _):
            acc[pl.ds(i * L_, L_)] = zf16
            acc2[pl.ds(i * L_, L_)] = zf16
            return 0

        lax.fori_loop(0, (L_ * astride) // L_, zero_body, 0)

        def issue(i, slot, sem):
            cid = i * nw + wid

            @pl.when(cid < n_chunks)
            def _():
                pltpu.async_copy(d_hbm.at[pl.ds(cid * CR_, CR_)],
                                 dbuf.at[slot], sem)
                pltpu.async_copy(f_hbm.at[pl.ds(cid * CR_, CR_)],
                                 fbuf.at[slot], sem)
                pltpu.async_copy(ei_hbm.at[:, pl.ds(cid * cedges, cedges)],
                                 ibuf.at[slot], sem)

        def drain(i, slot, sem):
            cid = i * nw + wid

            @pl.when(cid < n_chunks)
            def _():
                pltpu.make_async_copy(d_hbm.at[pl.ds(cid * CR_, CR_)],
                                      dbuf.at[slot], sem).wait()
                pltpu.make_async_copy(f_hbm.at[pl.ds(cid * CR_, CR_)],
                                      fbuf.at[slot], sem).wait()
                pltpu.make_async_copy(ei_hbm.at[:, pl.ds(cid * cedges, cedges)],
                                      ibuf.at[slot], sem).wait()

        def process(i, slot):
            cid = i * nw + wid

            @pl.when(cid < n_chunks)
            def _():
                @plsc.parallel_loop(0, CR_, step=1, unroll=4)
                def row_body(r):
                    for j in range(BK_ // L_):
                        o = j * L_
                        dx = dbuf[slot, r, pl.ds(o, L_)]
                        dy = dbuf[slot, r, pl.ds(BK_ + o, L_)]
                        dz = dbuf[slot, r, pl.ds(2 * BK_ + o, L_)]
                        fx = fbuf[slot, r, pl.ds(o, L_)]
                        fy = fbuf[slot, r, pl.ds(BK_ + o, L_)]
                        fz = fbuf[slot, r, pl.ds(2 * BK_ + o, L_)]
                        i0 = ibuf[slot, 0, pl.ds(r * BK_ + o, L_)]
                        i1 = ibuf[slot, 1, pl.ds(r * BK_ + o, L_)]
                        w0 = plsc.load_gather(btbl, [lax.shift_right_logical(i0, 1)])
                        w1 = plsc.load_gather(btbl, [lax.shift_right_logical(i1, 1)])
                        g0 = lax.shift_right_logical(
                            w0, lax.shift_left(i0 & 1, 4)) & 0xFFFF
                        g1 = lax.shift_right_logical(
                            w1, lax.shift_left(i1 & 1, 4)) & 0xFFFF
                        b0 = lane_base + g0
                        b1 = lane_base + g1
                        c = 0
                        for dc in (dx, dy, dz):
                            for fc in (fx, fy, fz):
                                p = dc * fc
                                sub = acc.at[pl.ds(c * 64, L_ * astride - c * 64)]
                                sub2 = acc2.at[pl.ds(c * 64, L_ * astride - c * 64)]
                                plsc.addupdate_scatter(sub, [b0], p)
                                plsc.addupdate_scatter(sub2, [b1], p)
                                c += 1

        issue(0, 0, sem0)

        def chunk_body(i, _):
            slot = lax.rem(i, 2)

            @pl.when(slot == 0)
            def _():
                issue(i + 1, 1, sem1)
                drain(i, 0, sem0)
                process(i, 0)

            @pl.when(slot == 1)
            def _():
                issue(i + 1, 0, sem0)
                drain(i, 1, sem1)
                process(i, 1)

            return 0

        lax.fori_loop(0, iters, chunk_body, 0)

        def red_body(j, _):
            s = zf16
            for lane in range(L_):
                off = lane * astride + j * L_
                s = s + acc[pl.ds(off, L_)] + acc2[pl.ds(off, L_)]
            orow[pl.ds(j * L_, L_)] = s
            return 0

        lax.fori_loop(0, nbins // L_, red_body, 0)
        pltpu.sync_copy(orow, out_hbm.at[pl.ds(wid * nbins, nbins)])

    return k(d_rows, f_rows, edge_index, bpacked)


def _pack_batch(batch):
    b = batch.astype(jnp.int32).reshape(-1, 2)
    return b[:, 0] | (b[:, 1] << 16)


def _to_block_rows(x):
    nb = x.shape[0] // BK_
    return jnp.pad(x.reshape(nb, BK_, 3), ((0, 0), (0, 0), (0, 1))) \
              .swapaxes(1, 2).reshape(nb, 4 * BK_)


def kernel(disp, pairwise_force, edge_index, batch):
    n_edges = disp.shape[0]
    n_nodes = batch.shape[0]
    partials = _sc_virial(
        _to_block_rows(disp.astype(jnp.float32)),
        _to_block_rows(pairwise_force.astype(jnp.float32)),
        edge_index.astype(jnp.int32),
        _pack_batch(batch),
        n_edges=n_edges, n_nodes=n_nodes)
    out9g = partials.reshape(-1, 9, N_GRAPHS_).sum(axis=0)
    return out9g.T.reshape(N_GRAPHS_, 3, 3)
